# baseline (speedup 1.0000x reference)
import functools

import jax
import jax.numpy as jnp
from jax import lax
from jax.experimental import pallas as pl
from jax.experimental.pallas import tpu as pltpu
from jax.experimental.pallas import tpu_sc as plsc

BATCH = 1024
SEQ = 77
HIDDEN = 768
ROWS = BATCH * SEQ
NUM_WORKERS = 32
WB = BATCH // NUM_WORKERS
IDS_PER_WORKER = WB * SEQ
LANES = 16
NPAIR = SEQ // 2


def _gather_pair(tok_hbm, idx_v, p, buf, h, sem):
    return pltpu.async_copy(
        tok_hbm.at[idx_v.at[pl.ds(p * 2 * WB, 2 * WB)]],
        buf.at[pl.ds(h * 2 * WB, 2 * WB)], sem)


def _wait_scatter(out_hbm, buf, wb0, sem):
    pltpu.make_async_copy(
        buf.at[pl.ds(0, WB)], out_hbm.at[pl.ds(wb0, WB)], sem).wait()


def _add_pos(posbuf, pq, buf, row0):
    def g_body(g, carry):
        o = g * 2 * LANES
        v0 = posbuf[pl.ds(pq * HIDDEN + o, LANES)]
        v1 = posbuf[pl.ds(pq * HIDDEN + o + LANES, LANES)]
        for i in range(WB):
            plsc.addupdate(buf.at[row0 + i, pl.ds(o, LANES)], v0)
            plsc.addupdate(buf.at[row0 + i, pl.ds(o + LANES, LANES)], v1)
        return carry

    lax.fori_loop(0, HIDDEN // (2 * LANES), g_body, 0)


def _emb_body(ids_hbm, tok_hbm, pos_hbm, out_hbm,
              idx_v, posbuf, buf, pos_sh, gsems, ssems):
    wid = lax.axis_index("s") * 2 + lax.axis_index("c")
    wb0 = wid * WB
    pltpu.sync_copy(ids_hbm.at[pl.ds(wid * IDS_PER_WORKER, IDS_PER_WORKER)],
                    idx_v)
    _gather_pair(tok_hbm, idx_v, 0, buf, 0, gsems[0])
    @pl.when(lax.axis_index("s") == 0)
    def _():
        pltpu.sync_copy(pos_hbm, pos_sh)
    plsc.subcore_barrier()

    def sub_body(p, h, i):
        hn = 1 - h

        def wait_and_prefetch():
            _wait_scatter(out_hbm, buf, wb0, ssems[hn])
            _wait_scatter(out_hbm, buf, wb0, ssems[hn])
            _gather_pair(tok_hbm, idx_v, p + 1, buf, hn, gsems[hn])

        if h == 0:
            @pl.when(i > 0)
            def _():
                wait_and_prefetch()

            @pl.when(i == 0)
            def _():
                _gather_pair(tok_hbm, idx_v, p + 1, buf, hn, gsems[hn])
        else:
            @pl.when(i < (NPAIR // 2) - 1)
            def _():
                wait_and_prefetch()

        pltpu.sync_copy(pos_sh.at[pl.ds(p * 2 * HIDDEN, 2 * HIDDEN)], posbuf)
        pltpu.make_async_copy(
            tok_hbm.at[idx_v.at[pl.ds(p * 2 * WB, 2 * WB)]],
            buf.at[pl.ds(h * 2 * WB, 2 * WB)], gsems[h]).wait()
        for pq in range(2):
            row0 = h * 2 * WB + pq * WB
            _add_pos(posbuf, pq, buf, row0)
            pltpu.async_copy(
                buf.at[pl.ds(row0, WB)],
                out_hbm.at[pl.ds((2 * p + pq) * BATCH + wb0, WB)], ssems[h])

    def duo_body(i, carry):
        sub_body(2 * i, 0, i)
        sub_body(2 * i + 1, 1, i)
        return carry

    lax.fori_loop(0, NPAIR // 2, duo_body, 0)

    c_t = SEQ - 1
    _wait_scatter(out_hbm, buf, wb0, ssems[0])
    _wait_scatter(out_hbm, buf, wb0, ssems[0])
    pltpu.async_copy(
        tok_hbm.at[idx_v.at[pl.ds(c_t * WB, WB)]],
        buf.at[pl.ds(0, WB)], gsems[0])
    pltpu.sync_copy(pos_sh.at[pl.ds(c_t * HIDDEN, HIDDEN)],
                    posbuf.at[pl.ds(0, HIDDEN)])
    pltpu.make_async_copy(
        tok_hbm.at[idx_v.at[pl.ds(c_t * WB, WB)]],
        buf.at[pl.ds(0, WB)], gsems[0]).wait()
    _add_pos(posbuf, 0, buf, 0)
    pltpu.async_copy(
        buf.at[pl.ds(0, WB)],
        out_hbm.at[pl.ds(c_t * BATCH + wb0, WB)], ssems[0])
    _wait_scatter(out_hbm, buf, wb0, ssems[1])
    _wait_scatter(out_hbm, buf, wb0, ssems[1])
    _wait_scatter(out_hbm, buf, wb0, ssems[0])


@functools.partial(
    pl.kernel,
    out_type=jax.ShapeDtypeStruct((ROWS, HIDDEN), jnp.float32),
    mesh=plsc.VectorSubcoreMesh(core_axis_name="c", subcore_axis_name="s"),
    scratch_types=[
        pltpu.VMEM((IDS_PER_WORKER,), jnp.int32),
        pltpu.VMEM((2 * HIDDEN,), jnp.float32),
        pltpu.VMEM((4 * WB, HIDDEN), jnp.float32),
        pltpu.VMEM_SHARED((SEQ * HIDDEN,), jnp.float32),
        pltpu.SemaphoreType.DMA,
        pltpu.SemaphoreType.DMA,
        pltpu.SemaphoreType.DMA,
        pltpu.SemaphoreType.DMA,
    ],
)
def _emb_kernel(ids_hbm, tok_hbm, pos_hbm, out_hbm,
                idx_v, posbuf, buf, pos_sh, g0, g1, s0, s1):
    _emb_body(ids_hbm, tok_hbm, pos_hbm, out_hbm,
              idx_v, posbuf, buf, pos_sh, (g0, g1), (s0, s1))


def kernel(input_ids, token_table, position_table):
    ids_grouped = (
        input_ids.astype(jnp.int32)
        .reshape(NUM_WORKERS, WB, SEQ)
        .transpose(0, 2, 1)
        .reshape(-1)
    )
    out = _emb_kernel(ids_grouped, token_table, position_table.reshape(-1))
    return out.reshape(SEQ, BATCH, HIDDEN).transpose(1, 0, 2)
